# baseline (speedup 1.0000x reference)
import functools

import jax
import jax.numpy as jnp
from jax import lax
from jax.experimental import pallas as pl
from jax.experimental.pallas import tpu as pltpu
from jax.experimental.pallas import tpu_sc as plsc

N_POINTS = 100000
C = 256
BLK = 16384
_GATHER_PAD = 8


def _sc_context_body(table_hbm, idx_hbm, bin_hbm, out_hbm, idx_v, rows_v, bin_v, sem):
    wid = lax.axis_index("s") * 2 + lax.axis_index("c")

    @pl.when(wid == 0)
    def _():
        pltpu.sync_copy(idx_hbm, idx_v)
        pltpu.async_copy(table_hbm.at[idx_v], rows_v, sem).wait()
        pltpu.sync_copy(bin_hbm, bin_v)
        for j in range(C // 16):
            sl = pl.ds(j * 16, 16)
            rows_v[0, sl] = rows_v[0, sl] + bin_v[sl]
        pltpu.sync_copy(rows_v.at[pl.ds(0, 1)], out_hbm)


def _sc_context(embedding_table, idx8, b_in):
    mesh = plsc.VectorSubcoreMesh(core_axis_name="c", subcore_axis_name="s")
    fn = functools.partial(
        pl.kernel,
        mesh=mesh,
        out_type=jax.ShapeDtypeStruct((1, C), jnp.float32),
        scratch_types=[
            pltpu.VMEM((_GATHER_PAD,), jnp.int32),
            pltpu.VMEM((_GATHER_PAD, C), jnp.float32),
            pltpu.VMEM((C,), jnp.float32),
            pltpu.SemaphoreType.DMA,
        ],
    )(_sc_context_body)
    return fn(embedding_table, idx8, b_in)


def _mlp_body(ctx_ref, coord_ref, win_ref, wout_ref, bout_ref, out_ref):
    h = (
        jax.lax.dot_general(
            coord_ref[...],
            win_ref[...],
            (((0,), (0,)), ((), ())),
            preferred_element_type=jnp.float32,
        )
        + ctx_ref[...]
    )
    h = jnp.maximum(h, 0.0)
    out_ref[...] = (
        jnp.dot(h, wout_ref[...], preferred_element_type=jnp.float32) + bout_ref[...]
    )


def kernel(coord, condition_idx, embedding_table, W_in, b_in, W_out, b_out):
    idx8 = jnp.broadcast_to(condition_idx.astype(jnp.int32), (_GATHER_PAD,))
    ctx = _sc_context(embedding_table, idx8, b_in)
    coord_t = coord.T
    return pl.pallas_call(
        _mlp_body,
        grid=((N_POINTS + BLK - 1) // BLK,),
        in_specs=[
            pl.BlockSpec((1, C), lambda i: (0, 0)),
            pl.BlockSpec((3, BLK), lambda i: (0, i)),
            pl.BlockSpec((3, C), lambda i: (0, 0)),
            pl.BlockSpec((C, C), lambda i: (0, 0)),
            pl.BlockSpec((1, C), lambda i: (0, 0)),
        ],
        out_specs=pl.BlockSpec((BLK, C), lambda i: (i, 0)),
        out_shape=jax.ShapeDtypeStruct((N_POINTS, C), jnp.float32),
        compiler_params=pltpu.CompilerParams(dimension_semantics=("arbitrary",)),
    )(ctx, coord_t, W_in, W_out, b_out.reshape(1, C))


# speedup vs baseline: 1.4534x; 1.4534x over previous
import jax
import jax.numpy as jnp
from jax.experimental import pallas as pl
from jax.experimental.pallas import tpu as pltpu

N_POINTS = 100000
C = 256
BLK = 16384


def _fused_body(idx_ref, tab_ref, bin_ref, coord_ref, win_ref, wout_ref, bout_ref, out_ref):
    del idx_ref
    ctx = tab_ref[0] + bin_ref[...]
    h = (
        jax.lax.dot_general(
            coord_ref[...],
            win_ref[...],
            (((0,), (0,)), ((), ())),
            preferred_element_type=jnp.float32,
        )
        + ctx
    )
    h = jnp.maximum(h, 0.0)
    out_ref[...] = (
        jnp.dot(h, wout_ref[...], preferred_element_type=jnp.float32) + bout_ref[...]
    )


def kernel(coord, condition_idx, embedding_table, W_in, b_in, W_out, b_out):
    idx = condition_idx.astype(jnp.int32)
    coord_t = coord.T
    grid_spec = pltpu.PrefetchScalarGridSpec(
        num_scalar_prefetch=1,
        grid=((N_POINTS + BLK - 1) // BLK,),
        in_specs=[
            pl.BlockSpec((1, 1, C), lambda i, idx: (idx[0], 0, 0)),
            pl.BlockSpec((1, C), lambda i, idx: (0, 0)),
            pl.BlockSpec((3, BLK), lambda i, idx: (0, i)),
            pl.BlockSpec((3, C), lambda i, idx: (0, 0)),
            pl.BlockSpec((C, C), lambda i, idx: (0, 0)),
            pl.BlockSpec((1, C), lambda i, idx: (0, 0)),
        ],
        out_specs=pl.BlockSpec((BLK, C), lambda i, idx: (i, 0)),
    )
    return pl.pallas_call(
        _fused_body,
        grid_spec=grid_spec,
        out_shape=jax.ShapeDtypeStruct((N_POINTS, C), jnp.float32),
        compiler_params=pltpu.CompilerParams(dimension_semantics=("parallel",)),
    )(
        idx,
        embedding_table.reshape(3, 1, C),
        b_in.reshape(1, C),
        coord_t,
        W_in,
        W_out,
        b_out.reshape(1, C),
    )
